# Initial kernel scaffold; baseline (speedup 1.0000x reference)
import jax
import jax.numpy as jnp
from jax.experimental import pallas as pl


def kernel(x, edge_index, W1, W2):
    raise NotImplementedError("write your pallas kernel here")


# speedup vs baseline: 3.5066x; 3.5066x over previous
import functools

import jax
import jax.numpy as jnp
from jax import lax
from jax.experimental import pallas as pl
from jax.experimental.pallas import tpu as pltpu
from jax.experimental.pallas import tpu_sc as plsc

N_NODES = 10000
N_FEAT = 256
N_EDGES = 160000
H_OUT = 128

NC = 2
NS = 16
CHUNK = 128
KROWS = 80
EPAD = KROWS * NS * CHUNK
NROW = 10240
RPT = NROW // NS
ZREP = RPT // CHUNK
RB = 400


def _degree_body(idx_hbm, out_hbm, idx_v, val_v, zv, acc):
    c = lax.axis_index("c")
    s = lax.axis_index("s")

    def fill_ones(i, carry):
        val_v[pl.ds(i * 16, 16)] = jnp.ones((16,), jnp.float32)
        return carry

    lax.fori_loop(0, CHUNK // 16, fill_ones, 0)

    def fill_zero(i, carry):
        zv[pl.ds(i * 16, 16)] = jnp.zeros((16,), jnp.float32)
        return carry

    lax.fori_loop(0, RPT // 16, fill_zero, 0)
    pltpu.sync_copy(zv, acc.at[pl.ds(s * RPT, RPT)])
    plsc.subcore_barrier()

    pltpu.sync_copy(idx_hbm.at[c].at[pl.ds(s * KROWS, KROWS)], idx_v)

    def body(j, carry):
        pltpu.sync_copy(val_v, acc.at[idx_v.at[j]], add=True)
        return carry

    lax.fori_loop(0, KROWS, body, 0)
    plsc.subcore_barrier()
    pltpu.sync_copy(acc.at[pl.ds(s * RPT, RPT)],
                    out_hbm.at[c].at[pl.ds(s * RPT, RPT)])


def _degrees(idx2):
    mesh = plsc.VectorSubcoreMesh(core_axis_name="c", subcore_axis_name="s")
    return pl.kernel(
        _degree_body,
        out_type=jax.ShapeDtypeStruct((NC, NROW), jnp.float32),
        mesh=mesh,
        scratch_types=[
            pltpu.VMEM((KROWS, CHUNK), jnp.int32),
            pltpu.VMEM((CHUNK,), jnp.float32),
            pltpu.VMEM((RPT,), jnp.float32),
            pltpu.VMEM_SHARED((NROW,), jnp.float32),
        ],
    )(idx2)


def _mp_body(table_hbm, srcg_hbm, dstp_hbm, out_hbm,
             idx_s, idx_d, gbuf, acc, sem, *, feature_split):
    c = lax.axis_index("c")
    s = lax.axis_index("s")

    def zrow(r, carry):
        for k in range(128 // 16):
            gbuf[r, pl.ds(k * 16, 16)] = jnp.zeros((16,), jnp.float32)
        return carry

    lax.fori_loop(0, CHUNK, zrow, 0)
    for z in range(ZREP):
        pltpu.sync_copy(gbuf, acc.at[pl.ds(s * RPT + z * CHUNK, CHUNK)])
    plsc.subcore_barrier()

    if feature_split:
        krows, base = KROWS, s * KROWS
        table = table_hbm.at[c]
    else:
        krows, base = KROWS // 2, (c * NS + s) * (KROWS // 2)
        table = table_hbm
    pltpu.sync_copy(srcg_hbm.at[pl.ds(base, krows)], idx_s.at[pl.ds(0, krows)])
    pltpu.sync_copy(dstp_hbm.at[pl.ds(base, krows)], idx_d.at[pl.ds(0, krows)])

    def body(j, carry):
        pltpu.async_copy(table.at[idx_s.at[j]], gbuf, sem).wait()
        pltpu.sync_copy(gbuf, acc.at[idx_d.at[j]], add=True)
        return carry

    lax.fori_loop(0, krows, body, 0)
    plsc.subcore_barrier()
    pltpu.sync_copy(acc.at[pl.ds(s * RPT, RPT)],
                    out_hbm.at[c].at[pl.ds(s * RPT, RPT)])


def _message_pass(table, srcg, dstp, feature_split):
    mesh = plsc.VectorSubcoreMesh(core_axis_name="c", subcore_axis_name="s")
    return pl.kernel(
        functools.partial(_mp_body, feature_split=feature_split),
        out_type=jax.ShapeDtypeStruct((NC, NROW, 128), jnp.float32),
        mesh=mesh,
        scratch_types=[
            pltpu.VMEM((KROWS, CHUNK), jnp.int32),
            pltpu.VMEM((KROWS, CHUNK), jnp.int32),
            pltpu.VMEM((CHUNK, 128), jnp.float32),
            pltpu.VMEM_SHARED((NROW, 128), jnp.float32),
            pltpu.SemaphoreType.DMA,
        ],
    )(table, srcg, dstp)


def _mm1_body(x_ref, deg_ref, w_ref, out_ref):
    s = lax.rsqrt(jnp.maximum(deg_ref[...], 1.0))
    out_ref[0] = jnp.dot(x_ref[...] * s, w_ref[...],
                         preferred_element_type=jnp.float32)


def _mm1(x, d_src, W1):
    return pl.pallas_call(
        _mm1_body,
        grid=(N_NODES // RB, 2),
        in_specs=[
            pl.BlockSpec((RB, N_FEAT), lambda i, c: (i, 0)),
            pl.BlockSpec((RB, 1), lambda i, c: (i, 0)),
            pl.BlockSpec((N_FEAT, 128), lambda i, c: (0, c)),
        ],
        out_specs=pl.BlockSpec((1, RB, 128), lambda i, c: (c, i, 0)),
        out_shape=jax.ShapeDtypeStruct((2, N_NODES, 128), jnp.float32),
    )(x, d_src, W1)


def _mm2_body(m1_ref, ds_ref, dd_ref, w_ref, out_ref):
    cs = (lax.rsqrt(jnp.maximum(ds_ref[...], 1.0))
          * lax.rsqrt(jnp.maximum(dd_ref[...], 1.0)))
    h0 = jnp.maximum(m1_ref[0], 0.0) * cs
    h1 = jnp.maximum(m1_ref[1], 0.0) * cs
    out_ref[...] = (
        jnp.dot(h0, w_ref[:128], preferred_element_type=jnp.float32)
        + jnp.dot(h1, w_ref[128:], preferred_element_type=jnp.float32))


def _mm2(m1, d_src, d_dst, W2):
    return pl.pallas_call(
        _mm2_body,
        grid=(N_NODES // RB,),
        in_specs=[
            pl.BlockSpec((2, RB, 128), lambda i: (0, i, 0)),
            pl.BlockSpec((RB, 1), lambda i: (i, 0)),
            pl.BlockSpec((RB, 1), lambda i: (i, 0)),
            pl.BlockSpec((N_FEAT, H_OUT), lambda i: (0, 0)),
        ],
        out_specs=pl.BlockSpec((RB, H_OUT), lambda i: (i, 0)),
        out_shape=jax.ShapeDtypeStruct((N_NODES, H_OUT), jnp.float32),
    )(m1, d_src, d_dst, W2)


def _fin_body(m2_ref, dd_ref, out_ref):
    si = lax.rsqrt(jnp.maximum(dd_ref[...], 1.0))
    out_ref[...] = (m2_ref[0] + m2_ref[1]) * si


def _fin(m2, d_dst):
    return pl.pallas_call(
        _fin_body,
        grid=(N_NODES // RB,),
        in_specs=[
            pl.BlockSpec((2, RB, H_OUT), lambda i: (0, i, 0)),
            pl.BlockSpec((RB, 1), lambda i: (i, 0)),
        ],
        out_specs=pl.BlockSpec((RB, H_OUT), lambda i: (i, 0)),
        out_shape=jax.ShapeDtypeStruct((N_NODES, H_OUT), jnp.float32),
    )(m2, d_dst)


def kernel(x, edge_index, W1, W2):
    src = edge_index[0].astype(jnp.int32)
    dst = edge_index[1].astype(jnp.int32)
    pad = EPAD - N_EDGES
    srcH = jnp.concatenate([src, jnp.full((pad,), N_NODES, jnp.int32)])
    srcG = jnp.concatenate([src, jnp.zeros((pad,), jnp.int32)])
    dstP = jnp.concatenate([dst, jnp.full((pad,), N_NODES, jnp.int32)])
    srcH = srcH.reshape(-1, CHUNK)
    srcG = srcG.reshape(-1, CHUNK)
    dstP = dstP.reshape(-1, CHUNK)
    idx2 = jnp.stack([srcH, dstP])

    hist = _degrees(idx2)
    d_src = hist[0].reshape(NROW, 1)
    d_dst = hist[1].reshape(NROW, 1)

    g1 = _mm1(x, d_src, W1)
    m1 = _message_pass(g1, srcG, dstP, True)
    g2 = _mm2(m1, d_src, d_dst, W2)
    m2 = _message_pass(g2, srcG, dstP, False)
    return _fin(m2, d_dst)


# speedup vs baseline: 3.6591x; 1.0435x over previous
import functools

import jax
import jax.numpy as jnp
from jax import lax
from jax.experimental import pallas as pl
from jax.experimental.pallas import tpu as pltpu
from jax.experimental.pallas import tpu_sc as plsc

N_NODES = 10000
N_FEAT = 256
N_EDGES = 160000
H_OUT = 128

NC = 2
NS = 16
CHUNK = 128
KROWS = 80
EPAD = KROWS * NS * CHUNK
NROW = 10240
RPT = NROW // NS
ZREP = RPT // CHUNK
RB = 400
IB = 8


def _degree_body(idx_hbm, out_hbm, idx_v, val_v, zv, acc):
    c = lax.axis_index("c")
    s = lax.axis_index("s")

    def fill_ones(i, carry):
        val_v[pl.ds(i * 16, 16)] = jnp.ones((16,), jnp.float32)
        return carry

    lax.fori_loop(0, CHUNK // 16, fill_ones, 0)

    def fill_zero(i, carry):
        zv[pl.ds(i * 16, 16)] = jnp.zeros((16,), jnp.float32)
        return carry

    lax.fori_loop(0, RPT // 16, fill_zero, 0)
    pltpu.sync_copy(zv, acc.at[pl.ds(s * RPT, RPT)])
    plsc.subcore_barrier()

    pltpu.sync_copy(idx_hbm.at[c].at[pl.ds(s * KROWS, KROWS)], idx_v)

    def body(j, carry):
        pltpu.sync_copy(val_v, acc.at[idx_v.at[j]], add=True)
        return carry

    lax.fori_loop(0, KROWS, body, 0)
    plsc.subcore_barrier()
    pltpu.sync_copy(acc.at[pl.ds(s * RPT, RPT)],
                    out_hbm.at[c].at[pl.ds(s * RPT, RPT)])


def _degrees(idx2):
    mesh = plsc.VectorSubcoreMesh(core_axis_name="c", subcore_axis_name="s")
    return pl.kernel(
        _degree_body,
        out_type=jax.ShapeDtypeStruct((NC, NROW), jnp.float32),
        mesh=mesh,
        scratch_types=[
            pltpu.VMEM((KROWS, CHUNK), jnp.int32),
            pltpu.VMEM((CHUNK,), jnp.float32),
            pltpu.VMEM((RPT,), jnp.float32),
            pltpu.VMEM_SHARED((NROW,), jnp.float32),
        ],
    )(idx2)


def _mp_body(table_hbm, srcg_hbm, dstp_hbm, out_hbm,
             idx_s, idx_d, gbuf, acc, sem0, sem1, *, feature_split):
    c = lax.axis_index("c")
    s = lax.axis_index("s")

    zb = gbuf.at[0]

    def zrow(r, carry):
        for k in range(128 // 16):
            zb[r, pl.ds(k * 16, 16)] = jnp.zeros((16,), jnp.float32)
        return carry

    lax.fori_loop(0, CHUNK, zrow, 0)
    for z in range(ZREP):
        pltpu.sync_copy(zb, acc.at[pl.ds(s * RPT + z * CHUNK, CHUNK)])
    plsc.subcore_barrier()

    if feature_split:
        krows, base = KROWS, s * KROWS
        table = table_hbm.at[c]
    else:
        krows, base = KROWS // 2, (c * NS + s) * (KROWS // 2)
        table = table_hbm
    sems = (sem0, sem1)

    def stage(t, carry):
        pltpu.sync_copy(srcg_hbm.at[pl.ds(base + t * IB, IB)], idx_s)
        pltpu.sync_copy(dstp_hbm.at[pl.ds(base + t * IB, IB)], idx_d)
        pltpu.async_copy(table.at[idx_s.at[0]], gbuf.at[0], sem0)

        def body(jj, carry2):
            for b in range(2):
                j = 2 * jj + b
                pltpu.make_async_copy(table.at[idx_s.at[j]],
                                      gbuf.at[b], sems[b]).wait()

                @pl.when(j + 1 < IB)
                def _():
                    pltpu.async_copy(table.at[idx_s.at[j + 1]],
                                     gbuf.at[1 - b], sems[1 - b])

                pltpu.sync_copy(gbuf.at[b], acc.at[idx_d.at[j]], add=True)
            return carry2

        lax.fori_loop(0, IB // 2, body, 0)
        return carry

    lax.fori_loop(0, krows // IB, stage, 0)
    plsc.subcore_barrier()
    pltpu.sync_copy(acc.at[pl.ds(s * RPT, RPT)],
                    out_hbm.at[c].at[pl.ds(s * RPT, RPT)])


def _message_pass(table, srcg, dstp, feature_split):
    mesh = plsc.VectorSubcoreMesh(core_axis_name="c", subcore_axis_name="s")
    return pl.kernel(
        functools.partial(_mp_body, feature_split=feature_split),
        out_type=jax.ShapeDtypeStruct((NC, NROW, 128), jnp.float32),
        mesh=mesh,
        scratch_types=[
            pltpu.VMEM((IB, CHUNK), jnp.int32),
            pltpu.VMEM((IB, CHUNK), jnp.int32),
            pltpu.VMEM((2, CHUNK, 128), jnp.float32),
            pltpu.VMEM_SHARED((NROW, 128), jnp.float32),
            pltpu.SemaphoreType.DMA,
            pltpu.SemaphoreType.DMA,
        ],
    )(table, srcg, dstp)


def _mm1_body(x_ref, deg_ref, w_ref, out_ref):
    s = lax.rsqrt(jnp.maximum(deg_ref[...], 1.0))
    out_ref[0] = jnp.dot(x_ref[...] * s, w_ref[...],
                         preferred_element_type=jnp.float32)


def _mm1(x, d_src, W1):
    return pl.pallas_call(
        _mm1_body,
        grid=(N_NODES // RB, 2),
        in_specs=[
            pl.BlockSpec((RB, N_FEAT), lambda i, c: (i, 0)),
            pl.BlockSpec((RB, 1), lambda i, c: (i, 0)),
            pl.BlockSpec((N_FEAT, 128), lambda i, c: (0, c)),
        ],
        out_specs=pl.BlockSpec((1, RB, 128), lambda i, c: (c, i, 0)),
        out_shape=jax.ShapeDtypeStruct((2, N_NODES, 128), jnp.float32),
    )(x, d_src, W1)


def _mm2_body(m1_ref, ds_ref, dd_ref, w_ref, out_ref):
    cs = (lax.rsqrt(jnp.maximum(ds_ref[...], 1.0))
          * lax.rsqrt(jnp.maximum(dd_ref[...], 1.0)))
    h0 = jnp.maximum(m1_ref[0], 0.0) * cs
    h1 = jnp.maximum(m1_ref[1], 0.0) * cs
    out_ref[...] = (
        jnp.dot(h0, w_ref[:128], preferred_element_type=jnp.float32)
        + jnp.dot(h1, w_ref[128:], preferred_element_type=jnp.float32))


def _mm2(m1, d_src, d_dst, W2):
    return pl.pallas_call(
        _mm2_body,
        grid=(N_NODES // RB,),
        in_specs=[
            pl.BlockSpec((2, RB, 128), lambda i: (0, i, 0)),
            pl.BlockSpec((RB, 1), lambda i: (i, 0)),
            pl.BlockSpec((RB, 1), lambda i: (i, 0)),
            pl.BlockSpec((N_FEAT, H_OUT), lambda i: (0, 0)),
        ],
        out_specs=pl.BlockSpec((RB, H_OUT), lambda i: (i, 0)),
        out_shape=jax.ShapeDtypeStruct((N_NODES, H_OUT), jnp.float32),
    )(m1, d_src, d_dst, W2)


def _fin_body(m2_ref, dd_ref, out_ref):
    si = lax.rsqrt(jnp.maximum(dd_ref[...], 1.0))
    out_ref[...] = (m2_ref[0] + m2_ref[1]) * si


def _fin(m2, d_dst):
    return pl.pallas_call(
        _fin_body,
        grid=(N_NODES // RB,),
        in_specs=[
            pl.BlockSpec((2, RB, H_OUT), lambda i: (0, i, 0)),
            pl.BlockSpec((RB, 1), lambda i: (i, 0)),
        ],
        out_specs=pl.BlockSpec((RB, H_OUT), lambda i: (i, 0)),
        out_shape=jax.ShapeDtypeStruct((N_NODES, H_OUT), jnp.float32),
    )(m2, d_dst)


def kernel(x, edge_index, W1, W2):
    src = edge_index[0].astype(jnp.int32)
    dst = edge_index[1].astype(jnp.int32)
    pad = EPAD - N_EDGES
    srcH = jnp.concatenate([src, jnp.full((pad,), N_NODES, jnp.int32)])
    srcG = jnp.concatenate([src, jnp.zeros((pad,), jnp.int32)])
    dstP = jnp.concatenate([dst, jnp.full((pad,), N_NODES, jnp.int32)])
    srcH = srcH.reshape(-1, CHUNK)
    srcG = srcG.reshape(-1, CHUNK)
    dstP = dstP.reshape(-1, CHUNK)
    idx2 = jnp.stack([srcH, dstP])

    hist = _degrees(idx2)
    d_src = hist[0].reshape(NROW, 1)
    d_dst = hist[1].reshape(NROW, 1)

    g1 = _mm1(x, d_src, W1)
    m1 = _message_pass(g1, srcG, dstP, True)
    g2 = _mm2(m1, d_src, d_dst, W2)
    m2 = _message_pass(g2, srcG, dstP, False)
    return _fin(m2, d_dst)


# speedup vs baseline: 3.6631x; 1.0011x over previous
import functools

import jax
import jax.numpy as jnp
from jax import lax
from jax.experimental import pallas as pl
from jax.experimental.pallas import tpu as pltpu
from jax.experimental.pallas import tpu_sc as plsc

N_NODES = 10000
N_FEAT = 256
N_EDGES = 160000
H_OUT = 128

NC = 2
NS = 16
CHUNK = 128
KROWS = 80
EPAD = KROWS * NS * CHUNK
NROW = 10240
RPT = NROW // NS
ZREP = RPT // CHUNK
RB = 400
IB = 8


def _degree_body(idx_hbm, out_hbm, idx_v, val_v, zv, acc):
    c = lax.axis_index("c")
    s = lax.axis_index("s")

    def fill_ones(i, carry):
        val_v[pl.ds(i * 16, 16)] = jnp.ones((16,), jnp.float32)
        return carry

    lax.fori_loop(0, CHUNK // 16, fill_ones, 0)

    def fill_zero(i, carry):
        zv[pl.ds(i * 16, 16)] = jnp.zeros((16,), jnp.float32)
        return carry

    lax.fori_loop(0, RPT // 16, fill_zero, 0)
    pltpu.sync_copy(zv, acc.at[pl.ds(s * RPT, RPT)])
    plsc.subcore_barrier()

    pltpu.sync_copy(idx_hbm.at[c].at[pl.ds(s * KROWS, KROWS)], idx_v)

    def body(j, carry):
        pltpu.sync_copy(val_v, acc.at[idx_v.at[j]], add=True)
        return carry

    lax.fori_loop(0, KROWS, body, 0)
    plsc.subcore_barrier()
    pltpu.sync_copy(acc.at[pl.ds(s * RPT, RPT)],
                    out_hbm.at[c].at[pl.ds(s * RPT, RPT)])


def _degrees(idx2):
    mesh = plsc.VectorSubcoreMesh(core_axis_name="c", subcore_axis_name="s")
    return pl.kernel(
        _degree_body,
        out_type=jax.ShapeDtypeStruct((NC, NROW), jnp.float32),
        mesh=mesh,
        scratch_types=[
            pltpu.VMEM((KROWS, CHUNK), jnp.int32),
            pltpu.VMEM((CHUNK,), jnp.float32),
            pltpu.VMEM((RPT,), jnp.float32),
            pltpu.VMEM_SHARED((NROW,), jnp.float32),
        ],
    )(idx2)


def _mp_body(table_hbm, srcg_hbm, dstp_hbm, out_hbm,
             idx_s, idx_d, gbuf, acc, gsem0, gsem1, ssem0, ssem1,
             *, feature_split):
    c = lax.axis_index("c")
    s = lax.axis_index("s")

    zb = gbuf.at[0]

    def zrow(r, carry):
        for k in range(128 // 16):
            zb[r, pl.ds(k * 16, 16)] = jnp.zeros((16,), jnp.float32)
        return carry

    lax.fori_loop(0, CHUNK, zrow, 0)
    for z in range(ZREP):
        pltpu.sync_copy(zb, acc.at[pl.ds(s * RPT + z * CHUNK, CHUNK)])
    plsc.subcore_barrier()

    if feature_split:
        krows, base = KROWS, s * KROWS
        table = table_hbm.at[c]
    else:
        krows, base = KROWS // 2, (c * NS + s) * (KROWS // 2)
        table = table_hbm
    gsems = (gsem0, gsem1)
    ssems = (ssem0, ssem1)

    def stage(t, carry):
        pltpu.sync_copy(srcg_hbm.at[pl.ds(base + t * IB, IB)], idx_s)
        pltpu.sync_copy(dstp_hbm.at[pl.ds(base + t * IB, IB)], idx_d)
        pltpu.async_copy(table.at[idx_s.at[0]], gbuf.at[0], gsem0)

        def body(jj, carry2):
            for b in range(2):
                j = 2 * jj + b
                pltpu.make_async_copy(table.at[idx_s.at[j]],
                                      gbuf.at[b], gsems[b]).wait()
                pltpu.async_copy(gbuf.at[b], acc.at[idx_d.at[j]],
                                 ssems[b], add=True)

                @pl.when(j >= 1)
                def _():
                    pltpu.make_async_copy(gbuf.at[1 - b],
                                          acc.at[idx_d.at[j]],
                                          ssems[1 - b]).wait()

                @pl.when(j + 1 < IB)
                def _():
                    pltpu.async_copy(table.at[idx_s.at[j + 1]],
                                     gbuf.at[1 - b], gsems[1 - b])
            return carry2

        lax.fori_loop(0, IB // 2, body, 0)
        pltpu.make_async_copy(gbuf.at[1], acc.at[idx_d.at[IB - 1]],
                              ssems[1]).wait()
        return carry

    lax.fori_loop(0, krows // IB, stage, 0)
    plsc.subcore_barrier()
    pltpu.sync_copy(acc.at[pl.ds(s * RPT, RPT)],
                    out_hbm.at[c].at[pl.ds(s * RPT, RPT)])


def _message_pass(table, srcg, dstp, feature_split):
    mesh = plsc.VectorSubcoreMesh(core_axis_name="c", subcore_axis_name="s")
    return pl.kernel(
        functools.partial(_mp_body, feature_split=feature_split),
        out_type=jax.ShapeDtypeStruct((NC, NROW, 128), jnp.float32),
        mesh=mesh,
        scratch_types=[
            pltpu.VMEM((IB, CHUNK), jnp.int32),
            pltpu.VMEM((IB, CHUNK), jnp.int32),
            pltpu.VMEM((2, CHUNK, 128), jnp.float32),
            pltpu.VMEM_SHARED((NROW, 128), jnp.float32),
            pltpu.SemaphoreType.DMA,
            pltpu.SemaphoreType.DMA,
            pltpu.SemaphoreType.DMA,
            pltpu.SemaphoreType.DMA,
        ],
    )(table, srcg, dstp)


def _mm1_body(x_ref, deg_ref, w_ref, out_ref):
    s = lax.rsqrt(jnp.maximum(deg_ref[...], 1.0))
    out_ref[0] = jnp.dot(x_ref[...] * s, w_ref[...],
                         preferred_element_type=jnp.float32)


def _mm1(x, d_src, W1):
    return pl.pallas_call(
        _mm1_body,
        grid=(N_NODES // RB, 2),
        in_specs=[
            pl.BlockSpec((RB, N_FEAT), lambda i, c: (i, 0)),
            pl.BlockSpec((RB, 1), lambda i, c: (i, 0)),
            pl.BlockSpec((N_FEAT, 128), lambda i, c: (0, c)),
        ],
        out_specs=pl.BlockSpec((1, RB, 128), lambda i, c: (c, i, 0)),
        out_shape=jax.ShapeDtypeStruct((2, N_NODES, 128), jnp.float32),
    )(x, d_src, W1)


def _mm2_body(m1_ref, ds_ref, dd_ref, w_ref, out_ref):
    cs = (lax.rsqrt(jnp.maximum(ds_ref[...], 1.0))
          * lax.rsqrt(jnp.maximum(dd_ref[...], 1.0)))
    h0 = jnp.maximum(m1_ref[0], 0.0) * cs
    h1 = jnp.maximum(m1_ref[1], 0.0) * cs
    out_ref[...] = (
        jnp.dot(h0, w_ref[:128], preferred_element_type=jnp.float32)
        + jnp.dot(h1, w_ref[128:], preferred_element_type=jnp.float32))


def _mm2(m1, d_src, d_dst, W2):
    return pl.pallas_call(
        _mm2_body,
        grid=(N_NODES // RB,),
        in_specs=[
            pl.BlockSpec((2, RB, 128), lambda i: (0, i, 0)),
            pl.BlockSpec((RB, 1), lambda i: (i, 0)),
            pl.BlockSpec((RB, 1), lambda i: (i, 0)),
            pl.BlockSpec((N_FEAT, H_OUT), lambda i: (0, 0)),
        ],
        out_specs=pl.BlockSpec((RB, H_OUT), lambda i: (i, 0)),
        out_shape=jax.ShapeDtypeStruct((N_NODES, H_OUT), jnp.float32),
    )(m1, d_src, d_dst, W2)


def _fin_body(m2_ref, dd_ref, out_ref):
    si = lax.rsqrt(jnp.maximum(dd_ref[...], 1.0))
    out_ref[...] = (m2_ref[0] + m2_ref[1]) * si


def _fin(m2, d_dst):
    return pl.pallas_call(
        _fin_body,
        grid=(N_NODES // RB,),
        in_specs=[
            pl.BlockSpec((2, RB, H_OUT), lambda i: (0, i, 0)),
            pl.BlockSpec((RB, 1), lambda i: (i, 0)),
        ],
        out_specs=pl.BlockSpec((RB, H_OUT), lambda i: (i, 0)),
        out_shape=jax.ShapeDtypeStruct((N_NODES, H_OUT), jnp.float32),
    )(m2, d_dst)


def kernel(x, edge_index, W1, W2):
    src = edge_index[0].astype(jnp.int32)
    dst = edge_index[1].astype(jnp.int32)
    pad = EPAD - N_EDGES
    srcH = jnp.concatenate([src, jnp.full((pad,), N_NODES, jnp.int32)])
    srcG = jnp.concatenate([src, jnp.zeros((pad,), jnp.int32)])
    dstP = jnp.concatenate([dst, jnp.full((pad,), N_NODES, jnp.int32)])
    srcH = srcH.reshape(-1, CHUNK)
    srcG = srcG.reshape(-1, CHUNK)
    dstP = dstP.reshape(-1, CHUNK)
    idx2 = jnp.stack([srcH, dstP])

    hist = _degrees(idx2)
    d_src = hist[0].reshape(NROW, 1)
    d_dst = hist[1].reshape(NROW, 1)

    g1 = _mm1(x, d_src, W1)
    m1 = _message_pass(g1, srcG, dstP, True)
    g2 = _mm2(m1, d_src, d_dst, W2)
    m2 = _message_pass(g2, srcG, dstP, False)
    return _fin(m2, d_dst)


# speedup vs baseline: 7.3713x; 2.0123x over previous
import functools

import jax
import jax.numpy as jnp
from jax import lax
from jax.experimental import pallas as pl
from jax.experimental.pallas import tpu as pltpu
from jax.experimental.pallas import tpu_sc as plsc

N_NODES = 10000
N_FEAT = 256
N_EDGES = 160000
H_OUT = 128

NC = 2
NS = 16
CHUNK = 128
KROWS = 80
EPAD = KROWS * NS * CHUNK
NROW = 10112
NBLK = NROW // CHUNK
NJUNK = NROW - N_NODES
NROWD = 10240
RPTD = NROWD // NS
RB = 400
IB = 8


def _degree_body(idx_hbm, out_hbm, idx_v, val_v, zv, acc):
    c = lax.axis_index("c")
    s = lax.axis_index("s")

    def fill_ones(i, carry):
        val_v[pl.ds(i * 16, 16)] = jnp.ones((16,), jnp.float32)
        return carry

    lax.fori_loop(0, CHUNK // 16, fill_ones, 0)

    def fill_zero(i, carry):
        zv[pl.ds(i * 16, 16)] = jnp.zeros((16,), jnp.float32)
        return carry

    lax.fori_loop(0, RPTD // 16, fill_zero, 0)
    pltpu.sync_copy(zv, acc.at[pl.ds(s * RPTD, RPTD)])
    plsc.subcore_barrier()

    pltpu.sync_copy(idx_hbm.at[c].at[pl.ds(s * KROWS, KROWS)], idx_v)

    def body(j, carry):
        pltpu.sync_copy(val_v, acc.at[idx_v.at[j]], add=True)
        return carry

    lax.fori_loop(0, KROWS, body, 0)
    plsc.subcore_barrier()
    pltpu.sync_copy(acc.at[pl.ds(s * RPTD, RPTD)],
                    out_hbm.at[c].at[pl.ds(s * RPTD, RPTD)])


def _degrees(idx2):
    mesh = plsc.VectorSubcoreMesh(core_axis_name="c", subcore_axis_name="s")
    return pl.kernel(
        _degree_body,
        out_type=jax.ShapeDtypeStruct((NC, NROWD), jnp.float32),
        mesh=mesh,
        scratch_types=[
            pltpu.VMEM((KROWS, CHUNK), jnp.int32),
            pltpu.VMEM((CHUNK,), jnp.float32),
            pltpu.VMEM((RPTD,), jnp.float32),
            pltpu.VMEM_SHARED((NROWD,), jnp.float32),
        ],
    )(idx2)


def _mp_body(table_hbm, srcg_hbm, dstp_hbm, out_hbm,
             idx_s, idx_d, gbuf, acc, gsem0, gsem1, ssem0, ssem1,
             *, feature_split):
    c = lax.axis_index("c")
    s = lax.axis_index("s")

    zb = gbuf.at[0]

    def zrow(r, carry):
        for k in range(128 // 16):
            zb[r, pl.ds(k * 16, 16)] = jnp.zeros((16,), jnp.float32)
        return carry

    lax.fori_loop(0, CHUNK, zrow, 0)
    for z in range((NBLK + NS - 1) // NS):
        blk = s + NS * z

        @pl.when(blk < NBLK)
        def _():
            pltpu.sync_copy(zb, acc.at[pl.ds(blk * CHUNK, CHUNK)])
    plsc.subcore_barrier()

    if feature_split:
        krows, base = KROWS, s * KROWS
        table = table_hbm.at[c]
    else:
        krows, base = KROWS // 2, (c * NS + s) * (KROWS // 2)
        table = table_hbm
    gsems = (gsem0, gsem1)
    ssems = (ssem0, ssem1)

    def stage(t, carry):
        pltpu.sync_copy(srcg_hbm.at[pl.ds(base + t * IB, IB)], idx_s)
        pltpu.sync_copy(dstp_hbm.at[pl.ds(base + t * IB, IB)], idx_d)
        pltpu.async_copy(table.at[idx_s.at[0]], gbuf.at[0], gsem0)

        def body(jj, carry2):
            for b in range(2):
                j = 2 * jj + b
                pltpu.make_async_copy(table.at[idx_s.at[j]],
                                      gbuf.at[b], gsems[b]).wait()
                pltpu.async_copy(gbuf.at[b], acc.at[idx_d.at[j]],
                                 ssems[b], add=True)

                @pl.when(j >= 1)
                def _():
                    pltpu.make_async_copy(gbuf.at[1 - b],
                                          acc.at[idx_d.at[j]],
                                          ssems[1 - b]).wait()

                @pl.when(j + 1 < IB)
                def _():
                    pltpu.async_copy(table.at[idx_s.at[j + 1]],
                                     gbuf.at[1 - b], gsems[1 - b])
            return carry2

        lax.fori_loop(0, IB // 2, body, 0)
        pltpu.make_async_copy(gbuf.at[1], acc.at[idx_d.at[IB - 1]],
                              ssems[1]).wait()
        return carry

    lax.fori_loop(0, krows // IB, stage, 0)
    plsc.subcore_barrier()
    for z in range((NBLK + NS - 1) // NS):
        blk = s + NS * z

        @pl.when(blk < NBLK)
        def _():
            pltpu.sync_copy(acc.at[pl.ds(blk * CHUNK, CHUNK)],
                            out_hbm.at[c].at[pl.ds(blk * CHUNK, CHUNK)])


def _message_pass(table, srcg, dstp, feature_split):
    mesh = plsc.VectorSubcoreMesh(core_axis_name="c", subcore_axis_name="s")
    return pl.kernel(
        functools.partial(_mp_body, feature_split=feature_split),
        out_type=jax.ShapeDtypeStruct((NC, NROW, 128), jnp.float32),
        mesh=mesh,
        scratch_types=[
            pltpu.VMEM((IB, CHUNK), jnp.int32),
            pltpu.VMEM((IB, CHUNK), jnp.int32),
            pltpu.VMEM((2, CHUNK, 128), jnp.float32),
            pltpu.VMEM_SHARED((NROW, 128), jnp.float32),
            pltpu.SemaphoreType.DMA,
            pltpu.SemaphoreType.DMA,
            pltpu.SemaphoreType.DMA,
            pltpu.SemaphoreType.DMA,
        ],
    )(table, srcg, dstp)


def _mm1_body(x_ref, deg_ref, w_ref, out_ref):
    s = lax.rsqrt(jnp.maximum(deg_ref[...], 1.0))
    out_ref[0] = jnp.dot(x_ref[...] * s, w_ref[...],
                         preferred_element_type=jnp.float32)


def _mm1(x, d_src, W1):
    return pl.pallas_call(
        _mm1_body,
        grid=(N_NODES // RB, 2),
        in_specs=[
            pl.BlockSpec((RB, N_FEAT), lambda i, c: (i, 0)),
            pl.BlockSpec((RB, 1), lambda i, c: (i, 0)),
            pl.BlockSpec((N_FEAT, 128), lambda i, c: (0, c)),
        ],
        out_specs=pl.BlockSpec((1, RB, 128), lambda i, c: (c, i, 0)),
        out_shape=jax.ShapeDtypeStruct((2, N_NODES, 128), jnp.float32),
    )(x, d_src, W1)


def _mm2_body(m1_ref, ds_ref, dd_ref, w_ref, out_ref):
    cs = (lax.rsqrt(jnp.maximum(ds_ref[...], 1.0))
          * lax.rsqrt(jnp.maximum(dd_ref[...], 1.0)))
    h0 = jnp.maximum(m1_ref[0], 0.0) * cs
    h1 = jnp.maximum(m1_ref[1], 0.0) * cs
    out_ref[...] = (
        jnp.dot(h0, w_ref[:128], preferred_element_type=jnp.float32)
        + jnp.dot(h1, w_ref[128:], preferred_element_type=jnp.float32))


def _mm2(m1, d_src, d_dst, W2):
    return pl.pallas_call(
        _mm2_body,
        grid=(N_NODES // RB,),
        in_specs=[
            pl.BlockSpec((2, RB, 128), lambda i: (0, i, 0)),
            pl.BlockSpec((RB, 1), lambda i: (i, 0)),
            pl.BlockSpec((RB, 1), lambda i: (i, 0)),
            pl.BlockSpec((N_FEAT, H_OUT), lambda i: (0, 0)),
        ],
        out_specs=pl.BlockSpec((RB, H_OUT), lambda i: (i, 0)),
        out_shape=jax.ShapeDtypeStruct((N_NODES, H_OUT), jnp.float32),
    )(m1, d_src, d_dst, W2)


def _fin_body(m2_ref, dd_ref, out_ref):
    si = lax.rsqrt(jnp.maximum(dd_ref[...], 1.0))
    out_ref[...] = (m2_ref[0] + m2_ref[1]) * si


def _fin(m2, d_dst):
    return pl.pallas_call(
        _fin_body,
        grid=(N_NODES // RB,),
        in_specs=[
            pl.BlockSpec((2, RB, H_OUT), lambda i: (0, i, 0)),
            pl.BlockSpec((RB, 1), lambda i: (i, 0)),
        ],
        out_specs=pl.BlockSpec((RB, H_OUT), lambda i: (i, 0)),
        out_shape=jax.ShapeDtypeStruct((N_NODES, H_OUT), jnp.float32),
    )(m2, d_dst)


def kernel(x, edge_index, W1, W2):
    src = edge_index[0].astype(jnp.int32)
    dst = edge_index[1].astype(jnp.int32)
    pad = EPAD - N_EDGES
    spread = jnp.arange(pad, dtype=jnp.int32)
    srcH = jnp.concatenate([src, N_NODES + spread % NJUNK])
    srcG = jnp.concatenate([src, spread % N_NODES])
    dstP = jnp.concatenate([dst, N_NODES + spread % NJUNK])
    srcH = srcH.reshape(-1, CHUNK)
    srcG = srcG.reshape(-1, CHUNK)
    dstP = dstP.reshape(-1, CHUNK)
    idx2 = jnp.stack([srcH, dstP])

    hist = _degrees(idx2)
    d_src = hist[0].reshape(NROWD, 1)
    d_dst = hist[1].reshape(NROWD, 1)

    g1 = _mm1(x, d_src, W1)
    m1 = _message_pass(g1, srcG, dstP, True)
    g2 = _mm2(m1, d_src, d_dst, W2)
    m2 = _message_pass(g2, srcG, dstP, False)
    return _fin(m2, d_dst)


# speedup vs baseline: 8.3905x; 1.1383x over previous
import functools

import jax
import jax.numpy as jnp
from jax import lax
from jax.experimental import pallas as pl
from jax.experimental.pallas import tpu as pltpu
from jax.experimental.pallas import tpu_sc as plsc

N_NODES = 10000
N_FEAT = 256
N_EDGES = 160000
H_OUT = 128

NC = 2
NS = 16
CHUNK = 128
KROWS = 80
EPAD = KROWS * NS * CHUNK
NROW = 10112
NBLK = NROW // CHUNK
NJUNK = NROW - N_NODES
NROWD = 10240
RPTD = NROWD // NS
RB = 1000
IB = 8


def _degree_body(idx_hbm, out_hbm, idx_v, val_v, zv, acc):
    c = lax.axis_index("c")
    s = lax.axis_index("s")

    def fill_ones(i, carry):
        val_v[pl.ds(i * 16, 16)] = jnp.ones((16,), jnp.float32)
        return carry

    lax.fori_loop(0, CHUNK // 16, fill_ones, 0)

    def fill_zero(i, carry):
        zv[pl.ds(i * 16, 16)] = jnp.zeros((16,), jnp.float32)
        return carry

    lax.fori_loop(0, RPTD // 16, fill_zero, 0)
    pltpu.sync_copy(zv, acc.at[pl.ds(s * RPTD, RPTD)])
    plsc.subcore_barrier()

    pltpu.sync_copy(idx_hbm.at[c].at[pl.ds(s * KROWS, KROWS)], idx_v)

    def body(j, carry):
        pltpu.sync_copy(val_v, acc.at[idx_v.at[j]], add=True)
        return carry

    lax.fori_loop(0, KROWS, body, 0)
    plsc.subcore_barrier()
    pltpu.sync_copy(acc.at[pl.ds(s * RPTD, RPTD)],
                    out_hbm.at[c].at[pl.ds(s * RPTD, RPTD)])


def _degrees(idx2):
    mesh = plsc.VectorSubcoreMesh(core_axis_name="c", subcore_axis_name="s")
    return pl.kernel(
        _degree_body,
        out_type=jax.ShapeDtypeStruct((NC, NROWD), jnp.float32),
        mesh=mesh,
        scratch_types=[
            pltpu.VMEM((KROWS, CHUNK), jnp.int32),
            pltpu.VMEM((CHUNK,), jnp.float32),
            pltpu.VMEM((RPTD,), jnp.float32),
            pltpu.VMEM_SHARED((NROWD,), jnp.float32),
        ],
    )(idx2)


def _mp_body(table_hbm, srcg_hbm, dstp_hbm, out_hbm,
             idx_s, idx_d, gbuf, acc, gsem0, gsem1, ssem0, ssem1,
             *, feature_split):
    c = lax.axis_index("c")
    s = lax.axis_index("s")

    zb = gbuf.at[0]

    def zrow(r, carry):
        for k in range(128 // 16):
            zb[r, pl.ds(k * 16, 16)] = jnp.zeros((16,), jnp.float32)
        return carry

    lax.fori_loop(0, CHUNK, zrow, 0)
    for z in range((NBLK + NS - 1) // NS):
        blk = s + NS * z

        @pl.when(blk < NBLK)
        def _():
            pltpu.sync_copy(zb, acc.at[pl.ds(blk * CHUNK, CHUNK)])
    plsc.subcore_barrier()

    if feature_split:
        krows, base = KROWS, s * KROWS
        table = table_hbm.at[c]
    else:
        krows, base = KROWS // 2, (c * NS + s) * (KROWS // 2)
        table = table_hbm
    gsems = (gsem0, gsem1)
    ssems = (ssem0, ssem1)

    def stage(t, carry):
        pltpu.sync_copy(srcg_hbm.at[pl.ds(base + t * IB, IB)], idx_s)
        pltpu.sync_copy(dstp_hbm.at[pl.ds(base + t * IB, IB)], idx_d)
        pltpu.async_copy(table.at[idx_s.at[0]], gbuf.at[0], gsem0)

        def body(jj, carry2):
            for b in range(2):
                j = 2 * jj + b
                pltpu.make_async_copy(table.at[idx_s.at[j]],
                                      gbuf.at[b], gsems[b]).wait()
                pltpu.async_copy(gbuf.at[b], acc.at[idx_d.at[j]],
                                 ssems[b], add=True)

                @pl.when(j >= 1)
                def _():
                    pltpu.make_async_copy(gbuf.at[1 - b],
                                          acc.at[idx_d.at[j]],
                                          ssems[1 - b]).wait()

                @pl.when(j + 1 < IB)
                def _():
                    pltpu.async_copy(table.at[idx_s.at[j + 1]],
                                     gbuf.at[1 - b], gsems[1 - b])
            return carry2

        lax.fori_loop(0, IB // 2, body, 0)
        pltpu.make_async_copy(gbuf.at[1], acc.at[idx_d.at[IB - 1]],
                              ssems[1]).wait()
        return carry

    lax.fori_loop(0, krows // IB, stage, 0)
    plsc.subcore_barrier()
    for z in range((NBLK + NS - 1) // NS):
        blk = s + NS * z

        @pl.when(blk < NBLK)
        def _():
            pltpu.sync_copy(acc.at[pl.ds(blk * CHUNK, CHUNK)],
                            out_hbm.at[c].at[pl.ds(blk * CHUNK, CHUNK)])


def _message_pass(table, srcg, dstp, feature_split):
    mesh = plsc.VectorSubcoreMesh(core_axis_name="c", subcore_axis_name="s")
    return pl.kernel(
        functools.partial(_mp_body, feature_split=feature_split),
        out_type=jax.ShapeDtypeStruct((NC, NROW, 128), jnp.float32),
        mesh=mesh,
        scratch_types=[
            pltpu.VMEM((IB, CHUNK), jnp.int32),
            pltpu.VMEM((IB, CHUNK), jnp.int32),
            pltpu.VMEM((2, CHUNK, 128), jnp.float32),
            pltpu.VMEM_SHARED((NROW, 128), jnp.float32),
            pltpu.SemaphoreType.DMA,
            pltpu.SemaphoreType.DMA,
            pltpu.SemaphoreType.DMA,
            pltpu.SemaphoreType.DMA,
        ],
    )(table, srcg, dstp)


def _mm1_body(x_ref, deg_ref, w_ref, out_ref):
    s = lax.rsqrt(jnp.maximum(deg_ref[...], 1.0))
    h = x_ref[...] * s
    out_ref[0] = jnp.dot(h, w_ref[:, :128], preferred_element_type=jnp.float32)
    out_ref[1] = jnp.dot(h, w_ref[:, 128:], preferred_element_type=jnp.float32)


def _mm1(x, d_src, W1):
    return pl.pallas_call(
        _mm1_body,
        grid=(N_NODES // RB,),
        in_specs=[
            pl.BlockSpec((RB, N_FEAT), lambda i: (i, 0)),
            pl.BlockSpec((RB, 1), lambda i: (i, 0)),
            pl.BlockSpec((N_FEAT, N_FEAT), lambda i: (0, 0)),
        ],
        out_specs=pl.BlockSpec((2, RB, 128), lambda i: (0, i, 0)),
        out_shape=jax.ShapeDtypeStruct((2, N_NODES, 128), jnp.float32),
    )(x, d_src, W1)


def _mm2_body(m1_ref, ds_ref, dd_ref, w_ref, out_ref):
    cs = (lax.rsqrt(jnp.maximum(ds_ref[...], 1.0))
          * lax.rsqrt(jnp.maximum(dd_ref[...], 1.0)))
    h0 = jnp.maximum(m1_ref[0], 0.0) * cs
    h1 = jnp.maximum(m1_ref[1], 0.0) * cs
    out_ref[...] = (
        jnp.dot(h0, w_ref[:128], preferred_element_type=jnp.float32)
        + jnp.dot(h1, w_ref[128:], preferred_element_type=jnp.float32))


def _mm2(m1, d_src, d_dst, W2):
    return pl.pallas_call(
        _mm2_body,
        grid=(N_NODES // RB,),
        in_specs=[
            pl.BlockSpec((2, RB, 128), lambda i: (0, i, 0)),
            pl.BlockSpec((RB, 1), lambda i: (i, 0)),
            pl.BlockSpec((RB, 1), lambda i: (i, 0)),
            pl.BlockSpec((N_FEAT, H_OUT), lambda i: (0, 0)),
        ],
        out_specs=pl.BlockSpec((RB, H_OUT), lambda i: (i, 0)),
        out_shape=jax.ShapeDtypeStruct((N_NODES, H_OUT), jnp.float32),
    )(m1, d_src, d_dst, W2)


def _fin_body(m2_ref, dd_ref, out_ref):
    si = lax.rsqrt(jnp.maximum(dd_ref[...], 1.0))
    out_ref[...] = (m2_ref[0] + m2_ref[1]) * si


def _fin(m2, d_dst):
    return pl.pallas_call(
        _fin_body,
        grid=(N_NODES // RB,),
        in_specs=[
            pl.BlockSpec((2, RB, H_OUT), lambda i: (0, i, 0)),
            pl.BlockSpec((RB, 1), lambda i: (i, 0)),
        ],
        out_specs=pl.BlockSpec((RB, H_OUT), lambda i: (i, 0)),
        out_shape=jax.ShapeDtypeStruct((N_NODES, H_OUT), jnp.float32),
    )(m2, d_dst)


def kernel(x, edge_index, W1, W2):
    src = edge_index[0].astype(jnp.int32)
    dst = edge_index[1].astype(jnp.int32)
    pad = EPAD - N_EDGES
    spread = jnp.arange(pad, dtype=jnp.int32)
    srcH = jnp.concatenate([src, N_NODES + spread % NJUNK])
    srcG = jnp.concatenate([src, spread % N_NODES])
    dstP = jnp.concatenate([dst, N_NODES + spread % NJUNK])
    srcH = srcH.reshape(-1, CHUNK)
    srcG = srcG.reshape(-1, CHUNK)
    dstP = dstP.reshape(-1, CHUNK)
    idx2 = jnp.stack([srcH, dstP])

    hist = _degrees(idx2)
    d_src = hist[0].reshape(NROWD, 1)
    d_dst = hist[1].reshape(NROWD, 1)

    g1 = _mm1(x, d_src, W1)
    m1 = _message_pass(g1, srcG, dstP, True)
    g2 = _mm2(m1, d_src, d_dst, W2)
    m2 = _message_pass(g2, srcG, dstP, False)
    return _fin(m2, d_dst)
